# baseline (speedup 1.0000x reference)
import functools

import jax
import jax.numpy as jnp
from jax import lax
from jax.experimental import pallas as pl
from jax.experimental.pallas import tpu as pltpu
from jax.experimental.pallas import tpu_sc as plsc

B, K = 16384, 100
NW = 32
ROWS_W = B // NW


@jax.jit
def _sc_gather(v, idx):
    mesh = plsc.VectorSubcoreMesh(core_axis_name="c", subcore_axis_name="s")

    @functools.partial(
        pl.kernel,
        mesh=mesh,
        out_type=jax.ShapeDtypeStruct((B, K), jnp.float32),
        scratch_types=[
            pltpu.VMEM_SHARED((1000000,), jnp.float32),
            pltpu.VMEM((128, K), jnp.int32),
            pltpu.VMEM((128, K), jnp.float32),
            pltpu.VMEM((20000,), jnp.float32),
            pltpu.SemaphoreType.DMA,
        ],
    )
    def k(v_hbm, idx_hbm, out_hbm, vs, idx_v, out_v, tmp, sem):
        sid = lax.axis_index("s")
        wid = sid * 2 + lax.axis_index("c")
        base = wid * ROWS_W

        with jax.named_scope("stage_v"):

            @pl.loop(sid, 50, step=16)
            def _stage(c):
                off = c * 20000
                pltpu.sync_copy(v_hbm.at[pl.ds(off, 20000)], tmp)
                pltpu.sync_copy(tmp, vs.at[pl.ds(off, 20000)])

            plsc.subcore_barrier()

        @pl.loop(0, 4)
        def _round(h):
            row0 = base + h * 128
            with jax.named_scope("idx_load"):
                pltpu.sync_copy(idx_hbm.at[pl.ds(row0, 128)], idx_v)

            with jax.named_scope("gather"):

                @pl.loop(0, 128)
                def _fire(r):
                    pltpu.async_copy(vs.at[idx_v.at[r]], out_v.at[r], sem)

                @pl.loop(0, 128)
                def _drain(r):
                    pltpu.make_async_copy(
                        vs.at[idx_v.at[r]], out_v.at[r], sem
                    ).wait()

            with jax.named_scope("out_store"):
                pltpu.sync_copy(out_v, out_hbm.at[pl.ds(row0, 128)])

    return k(v, idx)


def kernel(idx, v):
    return _sc_gather(v, idx.astype(jnp.int32))


# speedup vs baseline: 1.0021x; 1.0021x over previous
import functools

import jax
import jax.numpy as jnp
from jax import lax
from jax.experimental import pallas as pl
from jax.experimental.pallas import tpu as pltpu
from jax.experimental.pallas import tpu_sc as plsc

B, K = 16384, 100
NW = 32
ROWS_W = B // NW


@jax.jit
def _sc_gather(v, idx):
    mesh = plsc.VectorSubcoreMesh(core_axis_name="c", subcore_axis_name="s")

    @functools.partial(
        pl.kernel,
        mesh=mesh,
        out_type=jax.ShapeDtypeStruct((B, K), jnp.float32),
        compiler_params=pltpu.CompilerParams(use_tc_tiling_on_sc=True),
        scratch_types=[
            pltpu.VMEM_SHARED((1000000,), jnp.float32),
            pltpu.VMEM((128, K), jnp.int32),
            pltpu.VMEM((128, K), jnp.float32),
            pltpu.VMEM((20000,), jnp.float32),
            pltpu.SemaphoreType.DMA,
        ],
    )
    def k(v_hbm, idx_hbm, out_hbm, vs, idx_v, out_v, tmp, sem):
        sid = lax.axis_index("s")
        wid = sid * 2 + lax.axis_index("c")
        base = wid * ROWS_W

        with jax.named_scope("stage_v"):

            @pl.loop(sid, 50, step=16)
            def _stage(c):
                off = c * 20000
                pltpu.sync_copy(v_hbm.at[pl.ds(off, 20000)], tmp)
                pltpu.sync_copy(tmp, vs.at[pl.ds(off, 20000)])

            plsc.subcore_barrier()

        @pl.loop(0, 4)
        def _round(h):
            row0 = base + h * 128
            with jax.named_scope("idx_load"):
                pltpu.sync_copy(idx_hbm.at[pl.ds(row0, 128)], idx_v)

            with jax.named_scope("gather"):

                @pl.loop(0, 128)
                def _fire(r):
                    pltpu.async_copy(vs.at[idx_v.at[r]], out_v.at[r], sem)

                @pl.loop(0, 128)
                def _drain(r):
                    pltpu.make_async_copy(
                        vs.at[idx_v.at[r]], out_v.at[r], sem
                    ).wait()

            with jax.named_scope("out_store"):
                pltpu.sync_copy(out_v, out_hbm.at[pl.ds(row0, 128)])

    return k(v, idx)


def kernel(idx, v):
    return _sc_gather(v, idx.astype(jnp.int32))


# speedup vs baseline: 1.2902x; 1.2875x over previous
import functools

import jax
import jax.numpy as jnp
from jax import lax
from jax.experimental import pallas as pl
from jax.experimental.pallas import tpu as pltpu
from jax.experimental.pallas import tpu_sc as plsc

B, K = 16384, 100
NW = 32
COLS_W = B // NW
RC = 128
N_ROUNDS = COLS_W // RC


@jax.jit
def _sc_gather(v, idx_t):
    mesh = plsc.VectorSubcoreMesh(core_axis_name="c", subcore_axis_name="s")

    @functools.partial(
        pl.kernel,
        mesh=mesh,
        out_type=jax.ShapeDtypeStruct((K, B), jnp.float32),
        scratch_types=[
            pltpu.VMEM_SHARED((1000000,), jnp.float32),
            pltpu.VMEM((K, RC), jnp.int32),
            pltpu.VMEM((K, RC), jnp.float32),
            pltpu.VMEM((10000,), jnp.float32),
            pltpu.SemaphoreType.DMA,
        ],
    )
    def k(v_hbm, idx_hbm, out_hbm, vs, idx_v, out_v, tmp, sem):
        sid = lax.axis_index("s")
        wid = sid * 2 + lax.axis_index("c")

        with jax.named_scope("stage_v"):

            @pl.loop(sid, 100, step=16)
            def _stage(c):
                off = c * 10000
                pltpu.sync_copy(v_hbm.at[pl.ds(off, 10000)], tmp)
                pltpu.sync_copy(tmp, vs.at[pl.ds(off, 10000)])

            plsc.subcore_barrier()

        @pl.loop(0, N_ROUNDS)
        def _round(h):
            col0 = wid * COLS_W + h * RC
            with jax.named_scope("idx_load"):
                pltpu.sync_copy(idx_hbm.at[:, pl.ds(col0, RC)], idx_v)

            with jax.named_scope("gather"):

                @pl.loop(0, K)
                def _fire(r):
                    pltpu.async_copy(vs.at[idx_v.at[r]], out_v.at[r], sem)

                @pl.loop(0, K)
                def _drain(r):
                    pltpu.make_async_copy(
                        vs.at[idx_v.at[r]], out_v.at[r], sem
                    ).wait()

            with jax.named_scope("out_store"):
                pltpu.sync_copy(out_v, out_hbm.at[:, pl.ds(col0, RC)])

    return k(v, idx_t)


def kernel(idx, v):
    out_t = _sc_gather(v, idx.astype(jnp.int32).T)
    return out_t.T


# speedup vs baseline: 1.4868x; 1.1525x over previous
import functools

import jax
import jax.numpy as jnp
from jax import lax
from jax.experimental import pallas as pl
from jax.experimental.pallas import tpu as pltpu
from jax.experimental.pallas import tpu_sc as plsc

B, K = 16384, 100
NW = 32
COLS_W = B // NW
RC = 128
N_ROUNDS = COLS_W // RC
ST_CH = 5000
N_CH = 1000000 // ST_CH


@jax.jit
def _sc_gather(v, idx_t):
    mesh = plsc.VectorSubcoreMesh(core_axis_name="c", subcore_axis_name="s")

    @functools.partial(
        pl.kernel,
        mesh=mesh,
        out_type=jax.ShapeDtypeStruct((K, B), jnp.float32),
        scratch_types=[
            pltpu.VMEM_SHARED((1000000,), jnp.float32),
            pltpu.VMEM((K, RC), jnp.int32),
            pltpu.VMEM((K, RC), jnp.int32),
            pltpu.VMEM((K, RC), jnp.float32),
            pltpu.VMEM((K, RC), jnp.float32),
            pltpu.VMEM((ST_CH,), jnp.float32),
            pltpu.VMEM((ST_CH,), jnp.float32),
            pltpu.SemaphoreType.DMA,
            pltpu.SemaphoreType.DMA,
            pltpu.SemaphoreType.DMA,
            pltpu.SemaphoreType.DMA,
        ],
    )
    def k(v_hbm, idx_hbm, out_hbm, vs, idx_a, idx_b, out_a, out_b, tmp_a,
          tmp_b, sem_st, sem_i, sem_g, sem_o):
        sid = lax.axis_index("s")
        wid = sid * 2 + lax.axis_index("c")
        col0 = wid * COLS_W

        idx_bufs = [idx_a, idx_b]
        out_bufs = [out_a, out_b]

        def idx_slice(h):
            return idx_hbm.at[:, pl.ds(col0 + h * RC, RC)]

        def out_slice(h):
            return out_hbm.at[:, pl.ds(col0 + h * RC, RC)]

        pltpu.async_copy(idx_slice(0), idx_a, sem_i)

        with jax.named_scope("stage_v"):
            @pl.loop(sid, N_CH, step=32)
            def _stage(c):
                c2 = c + 16
                pltpu.async_copy(v_hbm.at[pl.ds(c * ST_CH, ST_CH)], tmp_a,
                                 sem_st)

                @pl.when(c2 < N_CH)
                def _():
                    pltpu.async_copy(v_hbm.at[pl.ds(c2 * ST_CH, ST_CH)],
                                     tmp_b, sem_st)

                pltpu.make_async_copy(v_hbm.at[pl.ds(c * ST_CH, ST_CH)],
                                      tmp_a, sem_st).wait()
                pltpu.sync_copy(tmp_a, vs.at[pl.ds(c * ST_CH, ST_CH)])

                @pl.when(c2 < N_CH)
                def _():
                    pltpu.make_async_copy(v_hbm.at[pl.ds(c2 * ST_CH, ST_CH)],
                                          tmp_b, sem_st).wait()
                    pltpu.sync_copy(tmp_b, vs.at[pl.ds(c2 * ST_CH, ST_CH)])

            plsc.subcore_barrier()

        for h in range(N_ROUNDS):
            cur_idx = idx_bufs[h % 2]
            cur_out = out_bufs[h % 2]
            with jax.named_scope("idx_wait"):
                pltpu.make_async_copy(idx_slice(h), cur_idx, sem_i).wait()
            if h + 1 < N_ROUNDS:
                pltpu.async_copy(idx_slice(h + 1), idx_bufs[(h + 1) % 2],
                                 sem_i)
            if h >= 2:
                with jax.named_scope("store_wait"):
                    pltpu.make_async_copy(cur_out, out_slice(h - 2),
                                          sem_o).wait()

            with jax.named_scope("gather"):

                @pl.loop(0, K)
                def _fire(r):
                    pltpu.async_copy(vs.at[cur_idx.at[r]], cur_out.at[r],
                                     sem_g)

                @pl.loop(0, K)
                def _drain(r):
                    pltpu.make_async_copy(vs.at[cur_idx.at[r]],
                                          cur_out.at[r], sem_g).wait()

            pltpu.async_copy(cur_out, out_slice(h), sem_o)

        with jax.named_scope("tail_waits"):
            pltpu.make_async_copy(out_bufs[(N_ROUNDS - 2) % 2],
                                  out_slice(N_ROUNDS - 2), sem_o).wait()
            pltpu.make_async_copy(out_bufs[(N_ROUNDS - 1) % 2],
                                  out_slice(N_ROUNDS - 1), sem_o).wait()

    return k(v, idx_t)


def kernel(idx, v):
    out_t = _sc_gather(v, idx.astype(jnp.int32).T)
    return out_t.T


# speedup vs baseline: 1.4966x; 1.0066x over previous
import functools

import jax
import jax.numpy as jnp
from jax import lax
from jax.experimental import pallas as pl
from jax.experimental.pallas import tpu as pltpu
from jax.experimental.pallas import tpu_sc as plsc

B, K = 16384, 100
NW = 32
COLS_W = B // NW
RC = 128
N_ROUNDS = COLS_W // RC
ST_CH = 5000
N_CH = 1000000 // ST_CH


@jax.jit
def _sc_gather(v, idx_t):
    mesh = plsc.VectorSubcoreMesh(core_axis_name="c", subcore_axis_name="s")

    @functools.partial(
        pl.kernel,
        mesh=mesh,
        out_type=jax.ShapeDtypeStruct((K, B), jnp.float32),
        scratch_types=[
            pltpu.VMEM_SHARED((1000000,), jnp.float32),
            pltpu.VMEM((K, RC), jnp.int32),
            pltpu.VMEM((K, RC), jnp.int32),
            pltpu.VMEM((K, RC), jnp.float32),
            pltpu.VMEM((K, RC), jnp.float32),
            pltpu.VMEM((ST_CH,), jnp.float32),
            pltpu.VMEM((ST_CH,), jnp.float32),
            pltpu.SemaphoreType.DMA,
            pltpu.SemaphoreType.DMA,
            pltpu.SemaphoreType.DMA,
            pltpu.SemaphoreType.DMA,
        ],
    )
    def k(v_hbm, idx_hbm, out_hbm, vs, idx_a, idx_b, out_a, out_b, tmp_a,
          tmp_b, sem_st, sem_i, sem_g, sem_o):
        sid = lax.axis_index("s")
        wid = sid * 2 + lax.axis_index("c")
        col0 = wid * COLS_W

        idx_bufs = [idx_a, idx_b]
        out_bufs = [out_a, out_b]

        def idx_slice(h):
            return idx_hbm.at[:, pl.ds(col0 + h * RC, RC)]

        def out_slice(h):
            return out_hbm.at[:, pl.ds(col0 + h * RC, RC)]

        pltpu.async_copy(idx_slice(0), idx_a, sem_i)

        with jax.named_scope("stage_v"):
            @pl.loop(sid, N_CH, step=32)
            def _stage(c):
                c2 = c + 16
                pltpu.async_copy(v_hbm.at[pl.ds(c * ST_CH, ST_CH)], tmp_a,
                                 sem_st)

                @pl.when(c2 < N_CH)
                def _():
                    pltpu.async_copy(v_hbm.at[pl.ds(c2 * ST_CH, ST_CH)],
                                     tmp_b, sem_st)

                pltpu.make_async_copy(v_hbm.at[pl.ds(c * ST_CH, ST_CH)],
                                      tmp_a, sem_st).wait()
                pltpu.sync_copy(tmp_a, vs.at[pl.ds(c * ST_CH, ST_CH)])

                @pl.when(c2 < N_CH)
                def _():
                    pltpu.make_async_copy(v_hbm.at[pl.ds(c2 * ST_CH, ST_CH)],
                                          tmp_b, sem_st).wait()
                    pltpu.sync_copy(tmp_b, vs.at[pl.ds(c2 * ST_CH, ST_CH)])

            plsc.subcore_barrier()

        for h in range(N_ROUNDS):
            cur_idx = idx_bufs[h % 2]
            cur_out = out_bufs[h % 2]
            with jax.named_scope("idx_wait"):
                pltpu.make_async_copy(idx_slice(h), cur_idx, sem_i).wait()
            if h + 1 < N_ROUNDS:
                pltpu.async_copy(idx_slice(h + 1), idx_bufs[(h + 1) % 2],
                                 sem_i)
            if h >= 2:
                with jax.named_scope("store_wait"):
                    pltpu.make_async_copy(cur_out, out_slice(h - 2),
                                          sem_o).wait()

            with jax.named_scope("gather"):

                @pl.loop(0, K, unroll=4)
                def _fire(r):
                    pltpu.async_copy(vs.at[cur_idx.at[r]], cur_out.at[r],
                                     sem_g)

                pltpu.make_async_copy(out_slice(h), cur_out, sem_g).wait()

            pltpu.async_copy(cur_out, out_slice(h), sem_o)

        with jax.named_scope("tail_waits"):
            pltpu.make_async_copy(out_bufs[(N_ROUNDS - 2) % 2],
                                  out_slice(N_ROUNDS - 2), sem_o).wait()
            pltpu.make_async_copy(out_bufs[(N_ROUNDS - 1) % 2],
                                  out_slice(N_ROUNDS - 1), sem_o).wait()

    return k(v, idx_t)


def kernel(idx, v):
    out_t = _sc_gather(v, idx.astype(jnp.int32).T)
    return out_t.T
